# Initial kernel scaffold; baseline (speedup 1.0000x reference)
import jax
import jax.numpy as jnp
from jax.experimental import pallas as pl


def kernel(x, A, B_ssm, W_imp, W_router, compress_neurons, knowledge_K, knowledge_V):
    raise NotImplementedError("write your pallas kernel here")


# speedup vs baseline: 16.1220x; 16.1220x over previous
import functools
import math

import jax
import jax.numpy as jnp
from jax import lax
from jax.experimental import pallas as pl
from jax.experimental.pallas import tpu as pltpu
from jax.experimental.pallas import tpu_sc as plsc

B, S, D = 2, 2048, 2048
RANK = 128
KR = 128
N_COMP = 64
N_KNOW = 16384
STATE = 64
TOPK = 8

CHUNK = 64
NCHUNK = S // CHUNK

NEG_INF = float("-inf")


def _eye(n):
    r = lax.broadcasted_iota(jnp.int32, (n, n), 0)
    c = lax.broadcasted_iota(jnp.int32, (n, n), 1)
    return jnp.where(r == c, 1.0, 0.0).astype(jnp.float32)


def _a_kernel(x_ref, w_ref, u_ref, pref_ref):
    xt = x_ref[0]
    r = jnp.dot(xt, w_ref[...], preferred_element_type=jnp.float32)
    u_ref[0] = r[:, :STATE]
    logits = r[:, STATE:]
    m = jnp.max(logits, axis=1, keepdims=True)
    e = jnp.exp(logits - m)
    pref_ref[0] = e / jnp.sum(e, axis=1, keepdims=True)


def _run_a(x, wcat):
    TS = 512
    return pl.pallas_call(
        _a_kernel,
        grid=(B, S // TS),
        in_specs=[
            pl.BlockSpec((1, TS, D), lambda b, s: (b, s, 0)),
            pl.BlockSpec((D, STATE + N_COMP), lambda b, s: (0, 0)),
        ],
        out_specs=[
            pl.BlockSpec((1, TS, STATE), lambda b, s: (b, s, 0)),
            pl.BlockSpec((1, TS, N_COMP), lambda b, s: (b, s, 0)),
        ],
        out_shape=[
            jax.ShapeDtypeStruct((B, S, STATE), jnp.float32),
            jax.ShapeDtypeStruct((B, S, N_COMP), jnp.float32),
        ],
    )(x, wcat)


_UNROLL = 8


def _b_kernel(u_ref, a_ref, wimp_ref, hproj_ref):
    a = a_ref[...]

    def body(i, h):
        for j in range(_UNROLL):
            h = (jnp.dot(h, a, preferred_element_type=jnp.float32)
                 + u_ref[i * _UNROLL + j])
        return h

    h0 = jnp.zeros((B, STATE), jnp.float32)
    h = lax.fori_loop(0, S // _UNROLL, body, h0)
    hproj_ref[...] = lax.dot_general(
        h, wimp_ref[...], (((1,), (1,)), ((), ())),
        preferred_element_type=jnp.float32,
    )


def _run_b(u, a, wimp):
    return pl.pallas_call(
        _b_kernel,
        in_specs=[
            pl.BlockSpec((S, B, STATE), lambda: (0, 0, 0)),
            pl.BlockSpec((STATE, STATE), lambda: (0, 0)),
            pl.BlockSpec((D, STATE), lambda: (0, 0)),
        ],
        out_specs=pl.BlockSpec((B, D), lambda: (0, 0)),
        out_shape=jax.ShapeDtypeStruct((B, D), jnp.float32),
    )(u, a, wimp)


def _c_kernel(x_ref, hp_ref, out_ref):
    xt = x_ref[...]
    hp = hp_ref[...]
    out_ref[...] = jnp.sum(xt * hp[:, None, :], axis=2)


def _run_c(x, hproj):
    TS = 512
    return pl.pallas_call(
        _c_kernel,
        grid=(S // TS,),
        in_specs=[
            pl.BlockSpec((B, TS, D), lambda s: (0, s, 0)),
            pl.BlockSpec((B, D), lambda s: (0, 0)),
        ],
        out_specs=pl.BlockSpec((B, TS), lambda s: (0, s)),
        out_shape=jax.ShapeDtypeStruct((B, S), jnp.float32),
    )(x, hproj)


def _d_kernel(il_ref, pref_ref, imp_ref, nw_ref):
    il = il_ref[...]
    m = jnp.max(il, axis=1, keepdims=True)
    e = jnp.exp(il - m)
    imp = e / jnp.sum(e, axis=1, keepdims=True)
    imp_ref[...] = imp
    rows = []
    for b in range(B):
        rows.append(
            jnp.dot(imp[b:b + 1, :], pref_ref[b],
                    preferred_element_type=jnp.float32)
        )
    nw = jnp.concatenate(rows, axis=0)
    nw_ref[...] = nw / (jnp.sum(nw, axis=1, keepdims=True) + 1e-8)


def _run_d(il, pref):
    return pl.pallas_call(
        _d_kernel,
        in_specs=[
            pl.BlockSpec((B, S), lambda: (0, 0)),
            pl.BlockSpec((B, S, N_COMP), lambda: (0, 0, 0)),
        ],
        out_specs=[
            pl.BlockSpec((B, S), lambda: (0, 0)),
            pl.BlockSpec((B, N_COMP), lambda: (0, 0)),
        ],
        out_shape=[
            jax.ShapeDtypeStruct((B, S), jnp.float32),
            jax.ShapeDtypeStruct((B, N_COMP), jnp.float32),
        ],
    )(il, pref)


def _e_kernel(nw_ref, cn_ref, out_ref):
    out_ref[...] = jnp.dot(nw_ref[...], cn_ref[...],
                           preferred_element_type=jnp.float32)


def _run_e(nw, cn_flat):
    TILE = 8192
    steps = (D * RANK) // TILE
    return pl.pallas_call(
        _e_kernel,
        grid=(steps,),
        in_specs=[
            pl.BlockSpec((B, N_COMP), lambda j: (0, 0)),
            pl.BlockSpec((N_COMP, TILE), lambda j: (0, j)),
        ],
        out_specs=pl.BlockSpec((B, TILE), lambda j: (0, j)),
        out_shape=jax.ShapeDtypeStruct((B, D * RANK), jnp.float32),
    )(nw, cn_flat)


_INV_SQRT_KR = 1.0 / math.sqrt(KR)
_F_TS = 256
_F_NCH = 4096


def _top8(work, ids, vacc, iacc):
    for _ in range(TOPK):
        m = jnp.max(work, axis=1, keepdims=True)
        sel = work == m
        ci = jnp.min(jnp.where(sel, ids, N_KNOW), axis=1, keepdims=True)
        vacc.append(m)
        iacc.append(ci)
        work = jnp.where(ids == ci, NEG_INF, work)
    return work


def _f_kernel(x_ref, sc_ref, k_ref, idx_ref, w_ref):
    q = jnp.dot(x_ref[0], sc_ref[0], preferred_element_type=jnp.float32)
    best_v = jnp.full((_F_TS, TOPK), NEG_INF, jnp.float32)
    best_i = jnp.zeros((_F_TS, TOPK), jnp.int32)
    for c in range(N_KNOW // _F_NCH):
        kc = k_ref[pl.ds(c * _F_NCH, _F_NCH), :]
        s_c = lax.dot_general(
            q, kc, (((1,), (1,)), ((), ())),
            preferred_element_type=jnp.float32,
        ) * _INV_SQRT_KR
        ids = lax.broadcasted_iota(jnp.int32, (_F_TS, _F_NCH), 1) + c * _F_NCH
        cv, ci = [], []
        _top8(s_c, ids, cv, ci)
        mv = jnp.concatenate([best_v] + cv, axis=1)
        mi = jnp.concatenate([best_i] + ci, axis=1)
        nv, ni = [], []
        _top8(mv, mi, nv, ni)
        best_v = jnp.concatenate(nv, axis=1)
        best_i = jnp.concatenate(ni, axis=1)
    e = jnp.exp(best_v - best_v[:, :1])
    w_ref[0] = e / jnp.sum(e, axis=1, keepdims=True)
    idx_ref[0] = best_i


def _run_f(x, shared_compress, knowledge_K):
    return pl.pallas_call(
        _f_kernel,
        grid=(B, S // _F_TS),
        in_specs=[
            pl.BlockSpec((1, _F_TS, D), lambda b, s: (b, s, 0)),
            pl.BlockSpec((1, D, RANK), lambda b, s: (b, 0, 0)),
            pl.BlockSpec((N_KNOW, KR), lambda b, s: (0, 0)),
        ],
        out_specs=[
            pl.BlockSpec((1, _F_TS, TOPK), lambda b, s: (b, s, 0)),
            pl.BlockSpec((1, _F_TS, TOPK), lambda b, s: (b, s, 0)),
        ],
        out_shape=[
            jax.ShapeDtypeStruct((B, S, TOPK), jnp.int32),
            jax.ShapeDtypeStruct((B, S, TOPK), jnp.float32),
        ],
    )(x, shared_compress, knowledge_K)


_TOK_PER_W = (B * S) // 32
_GT = 4
_NGROUP = _TOK_PER_W // _GT


def _g_kernel(v_hbm, idx_hbm, w_hbm, out_hbm, idx_v, w_v, rows_v, out_v, sem):
    wid = lax.axis_index("s") * 2 + lax.axis_index("c")
    base_tok = wid * _TOK_PER_W
    pltpu.sync_copy(idx_hbm.at[pl.ds(base_tok * TOPK, _TOK_PER_W * TOPK)], idx_v)
    pltpu.sync_copy(
        w_hbm.at[pl.ds(base_tok * TOPK * 16, _TOK_PER_W * TOPK * 16)], w_v)

    def group(g, carry):
        pltpu.async_copy(
            v_hbm.at[idx_v.at[pl.ds(g * (_GT * TOPK), _GT * TOPK)]],
            rows_v, sem,
        ).wait()
        for t in range(_GT):
            ws = [w_v[pl.ds((g * (_GT * TOPK) + t * TOPK + k) * 16, 16)]
                  for k in range(TOPK)]

            def jbody(j, c2, t=t, ws=ws):
                acc = ws[0] * rows_v[t * TOPK, pl.ds(j * 16, 16)]
                for k in range(1, TOPK):
                    acc = acc + ws[k] * rows_v[t * TOPK + k, pl.ds(j * 16, 16)]
                out_v[t, pl.ds(j * 16, 16)] = acc
                return c2

            lax.fori_loop(0, D // 16, jbody, 0)
        pltpu.sync_copy(out_v, out_hbm.at[pl.ds(base_tok + g * _GT, _GT)])
        return carry

    lax.fori_loop(0, _NGROUP, group, 0)


def _run_g(knowledge_V, idx_flat, w_splat):
    mesh = plsc.VectorSubcoreMesh(core_axis_name="c", subcore_axis_name="s")
    fn = functools.partial(
        pl.kernel,
        mesh=mesh,
        out_type=jax.ShapeDtypeStruct((B * S, D), jnp.float32),
        scratch_types=[
            pltpu.VMEM((_TOK_PER_W * TOPK,), jnp.int32),
            pltpu.VMEM((_TOK_PER_W * TOPK * 16,), jnp.float32),
            pltpu.VMEM((_GT * TOPK, D), jnp.float32),
            pltpu.VMEM((_GT, D), jnp.float32),
            pltpu.SemaphoreType.DMA,
        ],
    )(_g_kernel)
    return fn(knowledge_V, idx_flat, w_splat)


def kernel(x, A, B_ssm, W_imp, W_router, compress_neurons, knowledge_K,
           knowledge_V):
    x = x.astype(jnp.float32)
    wcat = jnp.concatenate([B_ssm, W_router.T], axis=1)
    u, pref = _run_a(x, wcat)
    hproj = _run_b(jnp.swapaxes(u, 0, 1), A, W_imp)

    il = _run_c(x, hproj)
    importance, nw = _run_d(il, pref)

    cn_flat = jnp.reshape(compress_neurons, (N_COMP, D * RANK))
    sc_flat = _run_e(nw, cn_flat)
    shared_compress = jnp.reshape(sc_flat, (B, D, RANK))

    topk_idx, weights = _run_f(x, shared_compress, knowledge_K)

    idx_flat = jnp.reshape(topk_idx, (B * S * TOPK,))
    w_splat = jnp.reshape(
        jnp.broadcast_to(jnp.reshape(weights, (B * S * TOPK, 1)),
                         (B * S * TOPK, 16)), (B * S * TOPK * 16,))
    out_flat = _run_g(knowledge_V, idx_flat, w_splat)
    output = jnp.reshape(out_flat, (B, S, D))

    return (output, importance, nw, topk_idx, weights)


# speedup vs baseline: 17.2485x; 1.0699x over previous
import functools
import math

import jax
import jax.numpy as jnp
from jax import lax
from jax.experimental import pallas as pl
from jax.experimental.pallas import tpu as pltpu
from jax.experimental.pallas import tpu_sc as plsc

B, S, D = 2, 2048, 2048
RANK = 128
KR = 128
N_COMP = 64
N_KNOW = 16384
STATE = 64
TOPK = 8

CHUNK = 64
NCHUNK = S // CHUNK

NEG_INF = float("-inf")


def _eye(n):
    r = lax.broadcasted_iota(jnp.int32, (n, n), 0)
    c = lax.broadcasted_iota(jnp.int32, (n, n), 1)
    return jnp.where(r == c, 1.0, 0.0).astype(jnp.float32)


def _a_kernel(x_ref, w_ref, u_ref, pref_ref):
    xt = x_ref[0]
    r = jnp.dot(xt, w_ref[...], preferred_element_type=jnp.float32)
    u_ref[0] = r[:, :STATE]
    logits = r[:, STATE:]
    m = jnp.max(logits, axis=1, keepdims=True)
    e = jnp.exp(logits - m)
    pref_ref[0] = e / jnp.sum(e, axis=1, keepdims=True)


def _run_a(x, wcat):
    TS = 512
    return pl.pallas_call(
        _a_kernel,
        grid=(B, S // TS),
        in_specs=[
            pl.BlockSpec((1, TS, D), lambda b, s: (b, s, 0)),
            pl.BlockSpec((D, STATE + N_COMP), lambda b, s: (0, 0)),
        ],
        out_specs=[
            pl.BlockSpec((1, TS, STATE), lambda b, s: (b, s, 0)),
            pl.BlockSpec((1, TS, N_COMP), lambda b, s: (b, s, 0)),
        ],
        out_shape=[
            jax.ShapeDtypeStruct((B, S, STATE), jnp.float32),
            jax.ShapeDtypeStruct((B, S, N_COMP), jnp.float32),
        ],
    )(x, wcat)


_UNROLL = 8


def _b_kernel(u_ref, a_ref, wimp_ref, hproj_ref):
    a = a_ref[...]

    def body(i, h):
        for j in range(_UNROLL):
            h = (jnp.dot(h, a, preferred_element_type=jnp.float32)
                 + u_ref[i * _UNROLL + j])
        return h

    h0 = jnp.zeros((B, STATE), jnp.float32)
    h = lax.fori_loop(0, S // _UNROLL, body, h0)
    hproj_ref[...] = lax.dot_general(
        h, wimp_ref[...], (((1,), (1,)), ((), ())),
        preferred_element_type=jnp.float32,
    )


def _run_b(u, a, wimp):
    return pl.pallas_call(
        _b_kernel,
        in_specs=[
            pl.BlockSpec((S, B, STATE), lambda: (0, 0, 0)),
            pl.BlockSpec((STATE, STATE), lambda: (0, 0)),
            pl.BlockSpec((D, STATE), lambda: (0, 0)),
        ],
        out_specs=pl.BlockSpec((B, D), lambda: (0, 0)),
        out_shape=jax.ShapeDtypeStruct((B, D), jnp.float32),
    )(u, a, wimp)


def _c_kernel(x_ref, hp_ref, out_ref):
    xt = x_ref[...]
    hp = hp_ref[...]
    out_ref[...] = jnp.sum(xt * hp[:, None, :], axis=2)


def _run_c(x, hproj):
    TS = 512
    return pl.pallas_call(
        _c_kernel,
        grid=(S // TS,),
        in_specs=[
            pl.BlockSpec((B, TS, D), lambda s: (0, s, 0)),
            pl.BlockSpec((B, D), lambda s: (0, 0)),
        ],
        out_specs=pl.BlockSpec((B, TS), lambda s: (0, s)),
        out_shape=jax.ShapeDtypeStruct((B, S), jnp.float32),
    )(x, hproj)


def _d_kernel(il_ref, pref_ref, imp_ref, nw_ref):
    il = il_ref[...]
    m = jnp.max(il, axis=1, keepdims=True)
    e = jnp.exp(il - m)
    imp = e / jnp.sum(e, axis=1, keepdims=True)
    imp_ref[...] = imp
    rows = []
    for b in range(B):
        rows.append(
            jnp.dot(imp[b:b + 1, :], pref_ref[b],
                    preferred_element_type=jnp.float32)
        )
    nw = jnp.concatenate(rows, axis=0)
    nw_ref[...] = nw / (jnp.sum(nw, axis=1, keepdims=True) + 1e-8)


def _run_d(il, pref):
    return pl.pallas_call(
        _d_kernel,
        in_specs=[
            pl.BlockSpec((B, S), lambda: (0, 0)),
            pl.BlockSpec((B, S, N_COMP), lambda: (0, 0, 0)),
        ],
        out_specs=[
            pl.BlockSpec((B, S), lambda: (0, 0)),
            pl.BlockSpec((B, N_COMP), lambda: (0, 0)),
        ],
        out_shape=[
            jax.ShapeDtypeStruct((B, S), jnp.float32),
            jax.ShapeDtypeStruct((B, N_COMP), jnp.float32),
        ],
    )(il, pref)


def _e_kernel(nw_ref, cn_ref, out_ref):
    out_ref[...] = jnp.dot(nw_ref[...], cn_ref[...],
                           preferred_element_type=jnp.float32)


def _run_e(nw, cn_flat):
    TILE = 8192
    steps = (D * RANK) // TILE
    return pl.pallas_call(
        _e_kernel,
        grid=(steps,),
        in_specs=[
            pl.BlockSpec((B, N_COMP), lambda j: (0, 0)),
            pl.BlockSpec((N_COMP, TILE), lambda j: (0, j)),
        ],
        out_specs=pl.BlockSpec((B, TILE), lambda j: (0, j)),
        out_shape=jax.ShapeDtypeStruct((B, D * RANK), jnp.float32),
    )(nw, cn_flat)


_INV_SQRT_KR = 1.0 / math.sqrt(KR)
_F_TS = 256
_F_NCH = 4096


def _top8(work, ids, vacc, iacc):
    for _ in range(TOPK):
        m = jnp.max(work, axis=1, keepdims=True)
        sel = work == m
        ci = jnp.min(jnp.where(sel, ids, N_KNOW), axis=1, keepdims=True)
        vacc.append(m)
        iacc.append(ci)
        work = jnp.where(ids == ci, NEG_INF, work)
    return work


def _f_kernel(x_ref, sc_ref, k_ref, idx_ref, w_ref):
    q = jnp.dot(x_ref[0], sc_ref[0], preferred_element_type=jnp.float32)
    best_v = jnp.full((_F_TS, TOPK), NEG_INF, jnp.float32)
    best_i = jnp.zeros((_F_TS, TOPK), jnp.int32)
    for c in range(N_KNOW // _F_NCH):
        kc = k_ref[pl.ds(c * _F_NCH, _F_NCH), :]
        s_c = lax.dot_general(
            q, kc, (((1,), (1,)), ((), ())),
            preferred_element_type=jnp.float32,
        ) * _INV_SQRT_KR
        ids = lax.broadcasted_iota(jnp.int32, (_F_TS, _F_NCH), 1) + c * _F_NCH
        cv, ci = [], []
        _top8(s_c, ids, cv, ci)
        mv = jnp.concatenate([best_v] + cv, axis=1)
        mi = jnp.concatenate([best_i] + ci, axis=1)
        nv, ni = [], []
        _top8(mv, mi, nv, ni)
        best_v = jnp.concatenate(nv, axis=1)
        best_i = jnp.concatenate(ni, axis=1)
    e = jnp.exp(best_v - best_v[:, :1])
    w_ref[0] = e / jnp.sum(e, axis=1, keepdims=True)
    idx_ref[0] = best_i


def _run_f(x, shared_compress, knowledge_K):
    return pl.pallas_call(
        _f_kernel,
        grid=(B, S // _F_TS),
        in_specs=[
            pl.BlockSpec((1, _F_TS, D), lambda b, s: (b, s, 0)),
            pl.BlockSpec((1, D, RANK), lambda b, s: (b, 0, 0)),
            pl.BlockSpec((N_KNOW, KR), lambda b, s: (0, 0)),
        ],
        out_specs=[
            pl.BlockSpec((1, _F_TS, TOPK), lambda b, s: (b, s, 0)),
            pl.BlockSpec((1, _F_TS, TOPK), lambda b, s: (b, s, 0)),
        ],
        out_shape=[
            jax.ShapeDtypeStruct((B, S, TOPK), jnp.int32),
            jax.ShapeDtypeStruct((B, S, TOPK), jnp.float32),
        ],
    )(x, shared_compress, knowledge_K)


_TOK_PER_W = (B * S) // 32
_GT = 2
_NGROUP = _TOK_PER_W // _GT


def _g_kernel(v_hbm, idx_hbm, w_hbm, out_hbm, idx_v, w_v, rows0, rows1,
              out_v, sem0, sem1):
    wid = lax.axis_index("s") * 2 + lax.axis_index("c")
    base_tok = wid * _TOK_PER_W
    pltpu.sync_copy(idx_hbm.at[pl.ds(base_tok * TOPK, _TOK_PER_W * TOPK)], idx_v)
    pltpu.sync_copy(
        w_hbm.at[pl.ds(base_tok * TOPK * 16, _TOK_PER_W * TOPK * 16)], w_v)

    def start(g, rows, sem):
        pltpu.make_async_copy(
            v_hbm.at[idx_v.at[pl.ds(g * (_GT * TOPK), _GT * TOPK)]],
            rows, sem,
        ).start()

    def compute(g, rows):
        for t in range(_GT):
            ws = [w_v[pl.ds((g * (_GT * TOPK) + t * TOPK + k) * 16, 16)]
                  for k in range(TOPK)]

            def jbody(j, c2, t=t, ws=ws):
                acc = ws[0] * rows[t * TOPK, pl.ds(j * 16, 16)]
                for k in range(1, TOPK):
                    acc = acc + ws[k] * rows[t * TOPK + k, pl.ds(j * 16, 16)]
                out_v[t, pl.ds(j * 16, 16)] = acc
                return c2

            lax.fori_loop(0, D // 16, jbody, 0)
        pltpu.sync_copy(out_v, out_hbm.at[pl.ds(base_tok + g * _GT, _GT)])

    start(0, rows0, sem0)
    start(1, rows1, sem1)

    def pair(gg, carry):
        for p, (rows, sem) in enumerate(((rows0, sem0), (rows1, sem1))):
            g = gg * 2 + p
            pltpu.make_async_copy(
                v_hbm.at[idx_v.at[pl.ds(g * (_GT * TOPK), _GT * TOPK)]],
                rows, sem,
            ).wait()
            compute(g, rows)

            @pl.when(g + 2 < _NGROUP)
            def _(g=g, rows=rows, sem=sem):
                start(g + 2, rows, sem)

        return carry

    lax.fori_loop(0, _NGROUP // 2, pair, 0)


def _run_g(knowledge_V, idx_flat, w_splat):
    mesh = plsc.VectorSubcoreMesh(core_axis_name="c", subcore_axis_name="s")
    fn = functools.partial(
        pl.kernel,
        mesh=mesh,
        out_type=jax.ShapeDtypeStruct((B * S, D), jnp.float32),
        scratch_types=[
            pltpu.VMEM((_TOK_PER_W * TOPK,), jnp.int32),
            pltpu.VMEM((_TOK_PER_W * TOPK * 16,), jnp.float32),
            pltpu.VMEM((_GT * TOPK, D), jnp.float32),
            pltpu.VMEM((_GT * TOPK, D), jnp.float32),
            pltpu.VMEM((_GT, D), jnp.float32),
            pltpu.SemaphoreType.DMA,
            pltpu.SemaphoreType.DMA,
        ],
    )(_g_kernel)
    return fn(knowledge_V, idx_flat, w_splat)


def kernel(x, A, B_ssm, W_imp, W_router, compress_neurons, knowledge_K,
           knowledge_V):
    x = x.astype(jnp.float32)
    wcat = jnp.concatenate([B_ssm, W_router.T], axis=1)
    u, pref = _run_a(x, wcat)
    hproj = _run_b(jnp.swapaxes(u, 0, 1), A, W_imp)

    il = _run_c(x, hproj)
    importance, nw = _run_d(il, pref)

    cn_flat = jnp.reshape(compress_neurons, (N_COMP, D * RANK))
    sc_flat = _run_e(nw, cn_flat)
    shared_compress = jnp.reshape(sc_flat, (B, D, RANK))

    topk_idx, weights = _run_f(x, shared_compress, knowledge_K)

    idx_flat = jnp.reshape(topk_idx, (B * S * TOPK,))
    w_splat = jnp.reshape(
        jnp.broadcast_to(jnp.reshape(weights, (B * S * TOPK, 1)),
                         (B * S * TOPK, 16)), (B * S * TOPK * 16,))
    out_flat = _run_g(knowledge_V, idx_flat, w_splat)
    output = jnp.reshape(out_flat, (B, S, D))

    return (output, importance, nw, topk_idx, weights)
